# baseline (speedup 1.0000x reference)
import functools

import jax
import jax.numpy as jnp
from jax import lax
from jax.experimental import pallas as pl
from jax.experimental.pallas import tpu as pltpu
from jax.experimental.pallas import tpu_sc as plsc

NC = 2
NS = 16
NW = NC * NS
LANES = 16
CHUNK = 48


def _make_sc_shuffle(B, C, D):
    rows_per_w = (B // NW) * C
    n_chunks = rows_per_w // CHUNK
    mesh = plsc.VectorSubcoreMesh(core_axis_name="c", subcore_axis_name="s")

    @functools.partial(
        pl.kernel,
        out_type=jax.ShapeDtypeStruct((B * C, D), jnp.float32),
        mesh=mesh,
        scratch_types=[
            pltpu.VMEM((C,), jnp.int32),
            pltpu.VMEM((rows_per_w,), jnp.int32),
            pltpu.VMEM((CHUNK, D), jnp.float32),
            pltpu.SemaphoreType.DMA,
        ],
    )
    def sc_shuffle(x_hbm, idx_hbm, out_hbm, idx_v, gidx_v, buf, sem):
        wid = lax.axis_index("s") * NC + lax.axis_index("c")
        base = wid * rows_per_w
        pltpu.sync_copy(idx_hbm, idx_v)

        def build_gidx(j, _):
            v = idx_v[pl.ds(j * LANES, LANES)]
            gidx_v[pl.ds(j * LANES, LANES)] = v + base
            gidx_v[pl.ds(C + j * LANES, LANES)] = v + base + C
            return 0

        lax.fori_loop(0, C // LANES, build_gidx, 0)

        def do_chunk(i, _):
            g = gidx_v.at[pl.ds(i * CHUNK, CHUNK)]
            pltpu.async_copy(x_hbm.at[g], buf, sem).wait()
            pltpu.sync_copy(buf, out_hbm.at[pl.ds(base + i * CHUNK, CHUNK)])
            return 0

        lax.fori_loop(0, n_chunks, do_chunk, 0)

    return sc_shuffle


def kernel(x, forward_shuffle_idx):
    B, C, H, W = x.shape
    D = H * W
    x2 = x.reshape(B * C, D)
    out = _make_sc_shuffle(B, C, D)(x2, forward_shuffle_idx)
    out = out.reshape(B, C, H, W)
    return (out, jnp.zeros((), x.dtype))


# speedup vs baseline: 1.0136x; 1.0136x over previous
import functools

import jax
import jax.numpy as jnp
from jax import lax
from jax.experimental import pallas as pl
from jax.experimental.pallas import tpu as pltpu
from jax.experimental.pallas import tpu_sc as plsc

NC = 2
NS = 16
NW = NC * NS
LANES = 16
CHUNK = 48


def _make_sc_shuffle(B, C, D):
    rows_per_w = (B // NW) * C
    n_chunks = rows_per_w // CHUNK
    n_pairs = n_chunks // 2
    mesh = plsc.VectorSubcoreMesh(core_axis_name="c", subcore_axis_name="s")

    @functools.partial(
        pl.kernel,
        out_type=jax.ShapeDtypeStruct((B * C, D), jnp.float32),
        mesh=mesh,
        scratch_types=[
            pltpu.VMEM((C,), jnp.int32),
            pltpu.VMEM((rows_per_w,), jnp.int32),
            pltpu.VMEM((CHUNK, D), jnp.float32),
            pltpu.VMEM((CHUNK, D), jnp.float32),
            pltpu.SemaphoreType.DMA,
            pltpu.SemaphoreType.DMA,
            pltpu.SemaphoreType.DMA,
            pltpu.SemaphoreType.DMA,
        ],
    )
    def sc_shuffle(x_hbm, idx_hbm, out_hbm, idx_v, gidx_v, buf0, buf1,
                   si0, si1, so0, so1):
        wid = lax.axis_index("s") * NC + lax.axis_index("c")
        base = wid * rows_per_w
        pltpu.sync_copy(idx_hbm, idx_v)

        def build_gidx(j, _):
            v = idx_v[pl.ds(j * LANES, LANES)]
            gidx_v[pl.ds(j * LANES, LANES)] = v + base
            gidx_v[pl.ds(C + j * LANES, LANES)] = v + base + C
            return 0

        lax.fori_loop(0, C // LANES, build_gidx, 0)

        def gather(c, buf, sem):
            return pltpu.make_async_copy(
                x_hbm.at[gidx_v.at[pl.ds(c * CHUNK, CHUNK)]], buf, sem)

        def put(c, buf, sem):
            return pltpu.make_async_copy(
                buf, out_hbm.at[pl.ds(base + c * CHUNK, CHUNK)], sem)

        gather(0, buf0, si0).start()
        gather(1, buf1, si1).start()

        def pair(p, _):
            c0 = 2 * p
            gather(c0, buf0, si0).wait()
            put(c0, buf0, so0).start()
            gather(c0 + 1, buf1, si1).wait()
            put(c0 + 1, buf1, so1).start()
            put(c0, buf0, so0).wait()
            gather(c0 + 2, buf0, si0).start()
            put(c0 + 1, buf1, so1).wait()
            gather(c0 + 3, buf1, si1).start()
            return 0

        lax.fori_loop(0, n_pairs - 1, pair, 0)

        c0 = n_chunks - 2
        gather(c0, buf0, si0).wait()
        put(c0, buf0, so0).start()
        gather(c0 + 1, buf1, si1).wait()
        put(c0 + 1, buf1, so1).start()
        put(c0, buf0, so0).wait()
        put(c0 + 1, buf1, so1).wait()

    return sc_shuffle


def kernel(x, forward_shuffle_idx):
    B, C, H, W = x.shape
    D = H * W
    x2 = x.reshape(B * C, D)
    out = _make_sc_shuffle(B, C, D)(x2, forward_shuffle_idx)
    out = out.reshape(B, C, H, W)
    return (out, jnp.zeros((), x.dtype))
